# baseline (speedup 1.0000x reference)
import functools
import math

import numpy as np
import jax
from jax import lax
import jax.numpy as jnp
from jax.experimental import pallas as pl
from jax.experimental.pallas import tpu as pltpu

_NT = 2048
_D = 2048
_NE = 16
_CAP = 256
_EPS = float(jnp.finfo(jnp.float32).eps)

_TB = 256
_NB = _NT // _TB
_CB = 128
_NCB = _NT // _CB


def _gumbel_const():
    return jax.random.gumbel(jax.random.key(42), (_NT, _NE), dtype=jnp.float32)


def _cumsum_lanes(m):
    s = 1
    while s < _NT:
        m = m + jnp.pad(m[:, :-s], ((0, 0), (s, 0)))
        s *= 2
    return m


def _first_argmax_rows(vals, e_iota):
    vmax = jnp.max(vals, axis=0, keepdims=True)
    return jnp.min(jnp.where(vals == vmax, e_iota, _NE), axis=0, keepdims=True)


_NS = 8


def _matmul_kernel(*refs):
    x_refs, wg_ref, out_ref = refs[:_NS], refs[_NS], refs[_NS + 1]
    q = _D // _NS
    w = wg_ref[...]
    acc = jnp.dot(x_refs[0][...], w[0:q], preferred_element_type=jnp.float32)
    for s in range(1, _NS):
        acc += jnp.dot(x_refs[s][...], w[s * q:(s + 1) * q],
                       preferred_element_type=jnp.float32)
    out_ref[...] = acc


def _routing_tables(lg_ref, gum_ref, laux_ref, lr_scr, wr_scr):
    logits = lg_ref[...].T
    lmax = jnp.max(logits, axis=0, keepdims=True)
    unnorm = jnp.exp(logits - lmax)
    gates = unnorm / jnp.sum(unnorm, axis=0, keepdims=True)

    e_iota = jax.lax.broadcasted_iota(jnp.int32, (_NE, _NT), 0)

    i1 = _first_argmax_rows(gates, e_iota)
    m1 = e_iota == i1
    mask1 = m1.astype(jnp.float32)

    noised = jnp.where(m1, -jnp.inf, logits + gum_ref[...])
    i2 = _first_argmax_rows(noised, e_iota)
    m2 = e_iota == i2
    mask2 = m2.astype(jnp.float32)

    cs1 = _cumsum_lanes(mask1)
    locations1 = cs1 - 1.0
    count1 = cs1[:, _NT - 1:_NT]
    locations2 = (_cumsum_lanes(mask2) - 1.0) + count1

    me = jnp.mean(gates, axis=1)
    ce = jnp.mean(mask1, axis=1)
    laux_ref[...] = (jnp.mean(me * ce) * (_NE * _NE)).reshape(1, 1)

    mask1 = mask1 * (locations1 < _CAP).astype(jnp.float32)
    mask2 = mask2 * (locations2 < _CAP).astype(jnp.float32)

    g1s = jnp.sum(gates * mask1, axis=0, keepdims=True)
    g2s = jnp.sum(gates * mask2, axis=0, keepdims=True)
    denom = jnp.maximum(g1s + g2s, _EPS)
    g1s = g1s / denom
    g2s = g2s / denom

    l1s = jnp.sum(locations1 * mask1, axis=0, keepdims=True).astype(jnp.int32)
    l2s = jnp.sum(locations2 * mask2, axis=0, keepdims=True).astype(jnp.int32)

    lr_t = jnp.where(m1, l1s, l2s)
    wr_t = g1s * mask1 + g2s * mask2
    lr_scr[...] = lr_t.T
    wr_scr[...] = wr_t.T


def _combine_kernel(lg_ref, gum_ref, cw_ref, dm_ref, laux_ref, lr_scr, wr_scr):
    i = pl.program_id(0)

    @pl.when(i == 0)
    def _():
        _routing_tables(lg_ref, gum_ref, laux_ref, lr_scr, wr_scr)

    lr = lr_scr[pl.ds(i * _CB, _CB), :]
    wr = wr_scr[pl.ds(i * _CB, _CB), :]
    c_iota = jax.lax.broadcasted_iota(jnp.int32, (_CB, _NE, _CAP), 2)
    oneh = c_iota == lr[:, :, None]
    cw = jnp.where(oneh, wr[:, :, None], 0.0)
    cw_ref[...] = cw
    dm_ref[...] = (cw > 0.0).astype(jnp.int8)


def kernel(x, Wg):
    gum_t = _gumbel_const().T

    q = _D // _NS
    logits = pl.pallas_call(
        _matmul_kernel,
        grid=(_NB,),
        in_specs=[
            pl.BlockSpec((_TB, q), functools.partial(lambda s, i: (i, s), s))
            for s in range(_NS)
        ] + [pl.BlockSpec((_D, _NE), lambda i: (0, 0))],
        out_specs=pl.BlockSpec((_TB, _NE), lambda i: (i, 0)),
        out_shape=jax.ShapeDtypeStruct((_NT, _NE), jnp.float32),
    )(*([x] * _NS), Wg)

    big = pl.BlockSpec((_CB, _NE, _CAP), lambda i: (i, 0, 0))
    cw, dm8, laux = pl.pallas_call(
        _combine_kernel,
        grid=(_NCB,),
        in_specs=[
            pl.BlockSpec((_NT, _NE), lambda i: (0, 0)),
            pl.BlockSpec((_NE, _NT), lambda i: (0, 0)),
        ],
        out_specs=[big, big, pl.BlockSpec((1, 1), lambda i: (0, 0))],
        out_shape=[
            jax.ShapeDtypeStruct((_NT, _NE, _CAP), jnp.float32),
            jax.ShapeDtypeStruct((_NT, _NE, _CAP), jnp.int8),
            jax.ShapeDtypeStruct((1, 1), jnp.float32),
        ],
        scratch_shapes=[
            pltpu.VMEM((_NT, _NE), jnp.int32),
            pltpu.VMEM((_NT, _NE), jnp.float32),
        ],
    )(logits, gum_t)

    dm = dm8.astype(jnp.bool_)
    return laux[0, 0], cw, dm
